# Initial kernel scaffold; baseline (speedup 1.0000x reference)
import jax
import jax.numpy as jnp
from jax.experimental import pallas as pl


def kernel(x, edge_index, W_mlp, b_mlp, W1, a1_src, a1_dst, g1, be1, W2, a2_src, a2_dst, g2, be2):
    raise NotImplementedError("write your pallas kernel here")


# speedup vs baseline: 6.0775x; 6.0775x over previous
import functools

import jax
import jax.numpy as jnp
from jax import lax
from jax.experimental import pallas as pl
from jax.experimental.pallas import tpu as pltpu
from jax.experimental.pallas import tpu_sc as plsc

_N = 10000
_E = 320000
_F_IN = 127
_H = 128
_C = 63

_NC = 2
_NS = 16
_NW = _NC * _NS

_BLK = 128
_PER_TILE = 10112
_NBLK = _PER_TILE // _BLK
_EPAD = _PER_TILE * _NW
_NPAD = 10112
_ROWS_PER_TILE = _NPAD // _NS


def _stage1_body(x_ref, wm_ref, b_ref, w1_ref, w2_ref,
                 a1s_ref, a1d_ref, a2s_ref, a2d_ref,
                 hcat_ref, st_ref, x3_ref, mm_ref):
    x0 = jnp.maximum(
        jnp.dot(x_ref[...], wm_ref[...], preferred_element_type=jnp.float32)
        + b_ref[...], 0.0)
    h1 = jnp.dot(x0, w1_ref[...], preferred_element_type=jnp.float32)
    h2 = jnp.dot(x0, w2_ref[...], preferred_element_type=jnp.float32)
    col = lax.broadcasted_iota(jnp.int32, (_NPAD, 64), 1)
    h1c = jnp.where(col == 63, 1.0, h1)
    h2c = jnp.where(col == 63, 1.0, h2)
    hcat_ref[...] = jnp.concatenate([h1c, h2c], axis=1)
    s1s = jnp.dot(h1, a1s_ref[...], preferred_element_type=jnp.float32)
    s1d = jnp.dot(h1, a1d_ref[...], preferred_element_type=jnp.float32)
    s2s = jnp.dot(h2, a2s_ref[...], preferred_element_type=jnp.float32)
    s2d = jnp.dot(h2, a2d_ref[...], preferred_element_type=jnp.float32)
    st_ref[...] = jnp.concatenate(
        [s1s, s1d, s2s, s2d, jnp.zeros((_NPAD, 12), jnp.float32)], axis=1)
    x3_ref[...] = jnp.max(x0, axis=1, keepdims=True)
    m1 = jnp.max(s1s) + jnp.max(s1d)
    m2 = jnp.max(s2s) + jnp.max(s2d)
    mm_ref[...] = jnp.stack(
        [jnp.full((16,), m1, jnp.float32), jnp.full((16,), m2, jnp.float32)])


def _sc_body(hcat_hbm, st_hbm, src_hbm, dst_hbm, zrows_hbm, mm_hbm, out_hbm,
             srcb, dstb, ssb, sdb, w1b, w2b, rows, m_v, acc, sem):
    c = lax.axis_index("c")
    s = lax.axis_index("s")
    wid = s * _NC + c

    pltpu.sync_copy(mm_hbm, m_v)
    pltpu.sync_copy(zrows_hbm, acc.at[pl.ds(s * _ROWS_PER_TILE, _ROWS_PER_TILE)])
    plsc.subcore_barrier()

    m1 = m_v[0]
    m2 = m_v[1]

    ebase = wid * _PER_TILE

    def block_body(b, carry):
        base = ebase + b * _BLK
        pltpu.sync_copy(src_hbm.at[pl.ds(base, _BLK)], srcb)
        pltpu.sync_copy(dst_hbm.at[pl.ds(base, _BLK)], dstb)
        gat = pltpu.async_copy(hcat_hbm.at[srcb], rows, sem)
        pltpu.sync_copy(st_hbm.at[srcb], ssb)
        pltpu.sync_copy(st_hbm.at[dstb], sdb)
        for j in range(_BLK // 16):
            sl = pl.ds(j * 16, 16)
            e16 = j * 16 + lax.iota(jnp.int32, 16)
            ss1 = plsc.load_gather(ssb, [e16, jnp.full((16,), 0, jnp.int32)])
            sd1 = plsc.load_gather(sdb, [e16, jnp.full((16,), 1, jnp.int32)])
            t1 = ss1 + sd1
            e1 = jnp.maximum(t1, 0.2 * t1)
            w1b[sl] = jnp.exp(e1 - m1)
            ss2 = plsc.load_gather(ssb, [e16, jnp.full((16,), 2, jnp.int32)])
            sd2 = plsc.load_gather(sdb, [e16, jnp.full((16,), 3, jnp.int32)])
            t2 = ss2 + sd2
            e2 = jnp.maximum(t2, 0.2 * t2)
            w2b[sl] = jnp.exp(e2 - m2)
        gat.wait()

        def scale_body(g, carry2):
            gs = pl.ds(g * 16, 16)
            w1v = w1b[gs]
            w2v = w2b[gs]
            ridx = g * 16 + lax.iota(jnp.int32, 16)
            for col16 in range(8):
                wv = w1v if col16 < 4 else w2v
                for k in range(16):
                    cidx = jnp.full((16,), col16 * 16 + k, jnp.int32)
                    v = plsc.load_gather(rows, [ridx, cidx]) * wv
                    plsc.store_scatter(rows, [ridx, cidx], v)
            return carry2

        lax.fori_loop(0, _BLK // 16, scale_body, 0)
        pltpu.sync_copy(rows, acc.at[dstb], add=True)
        return carry

    lax.fori_loop(0, _NBLK, block_body, 0)
    plsc.subcore_barrier()

    rsl = pl.ds(s * _ROWS_PER_TILE, _ROWS_PER_TILE)
    pltpu.sync_copy(acc.at[rsl], out_hbm.at[c, rsl])


def _stage2_body(acc_ref, x_ref, x3_ref, g1_ref, be1_ref, g2_ref, be2_ref,
                 out_ref):
    A = acc_ref[0] + acc_ref[1]

    def lngat_out(nsl, dsl, g, be):
        o = A[:, nsl] / (A[:, dsl] + 1e-16)
        mu = jnp.mean(o, axis=1, keepdims=True)
        d = o - mu
        var = jnp.mean(d * d, axis=1, keepdims=True)
        return d / jnp.sqrt(var + 1e-5) * g + be

    x1 = jnp.maximum(lngat_out(slice(0, 63), slice(63, 64),
                               g1_ref[...], be1_ref[...]), 0.0)
    x2 = lngat_out(slice(64, 127), slice(127, 128), g2_ref[...], be2_ref[...])
    y = jnp.concatenate([x1, x2, x3_ref[...]], axis=1) + x_ref[...]
    m = jnp.max(y, axis=1, keepdims=True)
    z = y - m
    out_ref[...] = z - jnp.log(jnp.sum(jnp.exp(z), axis=1, keepdims=True))


def kernel(x, edge_index, W_mlp, b_mlp, W1, a1_src, a1_dst, g1, be1,
           W2, a2_src, a2_dst, g2, be2):
    f32 = jnp.float32
    xp = jnp.zeros((_NPAD, _H), f32).at[:_N, :_F_IN].set(x)
    wmp = jnp.zeros((_H, _H), f32).at[:_F_IN, :].set(W_mlp)
    b2 = b_mlp.reshape(1, _H)
    w1p = jnp.zeros((_H, 64), f32).at[:, :_C].set(W1)
    w2p = jnp.zeros((_H, 64), f32).at[:, :_C].set(W2)
    a1s = jnp.zeros((64, 1), f32).at[:_C, 0].set(a1_src)
    a1d = jnp.zeros((64, 1), f32).at[:_C, 0].set(a1_dst)
    a2s = jnp.zeros((64, 1), f32).at[:_C, 0].set(a2_src)
    a2d = jnp.zeros((64, 1), f32).at[:_C, 0].set(a2_dst)

    hcat, st, x3, mm = pl.pallas_call(
        _stage1_body,
        out_shape=[
            jax.ShapeDtypeStruct((_NPAD, _H), f32),
            jax.ShapeDtypeStruct((_NPAD, 16), f32),
            jax.ShapeDtypeStruct((_NPAD, 1), f32),
            jax.ShapeDtypeStruct((2, 16), f32),
        ],
    )(xp, wmp, b2, w1p, w2p, a1s, a1d, a2s, a2d)

    src = jnp.concatenate(
        [edge_index[0], jnp.zeros((_EPAD - _E,), jnp.int32)])
    dst = jnp.concatenate(
        [edge_index[1], jnp.full((_EPAD - _E,), _N, jnp.int32)])
    zrows = jnp.zeros((_ROWS_PER_TILE, _H), f32)

    mesh = plsc.VectorSubcoreMesh(core_axis_name="c", subcore_axis_name="s")
    acc = pl.kernel(
        _sc_body,
        mesh=mesh,
        out_type=jax.ShapeDtypeStruct((_NC, _NPAD, _H), f32),
        compiler_params=pltpu.CompilerParams(
            needs_layout_passes=False, use_tc_tiling_on_sc=False),
        scratch_types=[
            pltpu.VMEM((_BLK,), jnp.int32),
            pltpu.VMEM((_BLK,), jnp.int32),
            pltpu.VMEM((_BLK, 16), f32),
            pltpu.VMEM((_BLK, 16), f32),
            pltpu.VMEM((_BLK,), f32),
            pltpu.VMEM((_BLK,), f32),
            pltpu.VMEM((_BLK, _H), f32),
            pltpu.VMEM((2, 16), f32),
            pltpu.VMEM_SHARED((_NPAD, _H), f32),
            pltpu.SemaphoreType.DMA,
        ],
    )(hcat, st, src, dst, zrows, mm)

    xres = jnp.zeros((_NPAD, _F_IN), f32).at[:_N, :].set(x)
    out = pl.pallas_call(
        _stage2_body,
        out_shape=jax.ShapeDtypeStruct((_NPAD, _F_IN), f32),
    )(acc, xres, x3, g1.reshape(1, _C), be1.reshape(1, _C),
      g2.reshape(1, _C), be2.reshape(1, _C))
    return out[:_N]


# speedup vs baseline: 29.6254x; 4.8746x over previous
import functools

import jax
import jax.numpy as jnp
from jax import lax
from jax.experimental import pallas as pl
from jax.experimental.pallas import tpu as pltpu
from jax.experimental.pallas import tpu_sc as plsc

_N = 10000
_E = 320000
_F_IN = 127
_H = 128
_C = 63

_NC = 2
_NS = 16
_NW = _NC * _NS

_BLK = 64
_PER_TILE = 10112
_NBLK = _PER_TILE // _BLK
_EPAD = _PER_TILE * _NW
_NPAD = 10112
_ROWS_PER_TILE = _NPAD // _NS


def _stage1_body(x_ref, wm_ref, b_ref, w1_ref, w2_ref,
                 a1s_ref, a1d_ref, a2s_ref, a2d_ref,
                 hcat_ref, st_ref, x3_ref, mm_ref):
    x0 = jnp.maximum(
        jnp.dot(x_ref[...], wm_ref[...], preferred_element_type=jnp.float32)
        + b_ref[...], 0.0)
    h1 = jnp.dot(x0, w1_ref[...], preferred_element_type=jnp.float32)
    h2 = jnp.dot(x0, w2_ref[...], preferred_element_type=jnp.float32)
    col = lax.broadcasted_iota(jnp.int32, (_NPAD, 64), 1)
    h1c = jnp.where(col == 63, 1.0, h1)
    h2c = jnp.where(col == 63, 1.0, h2)
    hcat_ref[...] = jnp.concatenate([h1c, h2c], axis=1)
    s1s = jnp.dot(h1, a1s_ref[...], preferred_element_type=jnp.float32)
    s1d = jnp.dot(h1, a1d_ref[...], preferred_element_type=jnp.float32)
    s2s = jnp.dot(h2, a2s_ref[...], preferred_element_type=jnp.float32)
    s2d = jnp.dot(h2, a2d_ref[...], preferred_element_type=jnp.float32)
    st_ref[...] = jnp.concatenate(
        [s1s, s1d, s2s, s2d, jnp.zeros((_NPAD, 12), jnp.float32)], axis=1)
    x3_ref[...] = jnp.max(x0, axis=1, keepdims=True)
    m1 = jnp.max(s1s) + jnp.max(s1d)
    m2 = jnp.max(s2s) + jnp.max(s2d)
    mm_ref[...] = jnp.stack(
        [jnp.full((16,), m1, jnp.float32), jnp.full((16,), m2, jnp.float32)])


def _sc_body(hcat_hbm, st_hbm, src_hbm, dst_hbm, zrows_hbm, mm_hbm, out_hbm,
             srcall, dstall, ssb0, ssb1, sdb0, sdb1, rows0, rows1,
             w1b, w2b, m_v, acc, sem_g0, sem_g1):
    c = lax.axis_index("c")
    s = lax.axis_index("s")
    wid = s * _NC + c

    ssb = (ssb0, ssb1)
    sdb = (sdb0, sdb1)
    rows = (rows0, rows1)
    sem_g = (sem_g0, sem_g1)

    pltpu.sync_copy(mm_hbm, m_v)
    pltpu.sync_copy(src_hbm.at[wid], srcall)
    pltpu.sync_copy(dst_hbm.at[wid], dstall)
    pltpu.sync_copy(zrows_hbm, acc.at[pl.ds(s * _ROWS_PER_TILE, _ROWS_PER_TILE)])
    plsc.subcore_barrier()

    m1 = m_v[0]
    m2 = m_v[1]

    def issue_gath(b, p):
        sidx = srcall.at[b]
        didx = dstall.at[b]
        pltpu.async_copy(st_hbm.at[sidx], ssb[p], sem_g[p])
        pltpu.async_copy(st_hbm.at[didx], sdb[p], sem_g[p])
        pltpu.async_copy(hcat_hbm.at[sidx], rows[p], sem_g[p])

    def compute(b, p):
        pltpu.make_async_copy(st_hbm.at[pl.ds(0, _BLK)], ssb[p], sem_g[p]).wait()
        pltpu.make_async_copy(st_hbm.at[pl.ds(0, _BLK)], sdb[p], sem_g[p]).wait()
        pltpu.make_async_copy(
            hcat_hbm.at[pl.ds(0, _BLK)], rows[p], sem_g[p]).wait()
        for j in range(_BLK // 16):
            sl = pl.ds(j * 16, 16)
            e16 = j * 16 + lax.iota(jnp.int32, 16)
            ss1 = plsc.load_gather(ssb[p], [e16, jnp.full((16,), 0, jnp.int32)])
            sd1 = plsc.load_gather(sdb[p], [e16, jnp.full((16,), 1, jnp.int32)])
            t1 = ss1 + sd1
            e1 = jnp.maximum(t1, 0.2 * t1)
            w1b[sl] = jnp.exp(e1 - m1)
            ss2 = plsc.load_gather(ssb[p], [e16, jnp.full((16,), 2, jnp.int32)])
            sd2 = plsc.load_gather(sdb[p], [e16, jnp.full((16,), 3, jnp.int32)])
            t2 = ss2 + sd2
            e2 = jnp.maximum(t2, 0.2 * t2)
            w2b[sl] = jnp.exp(e2 - m2)

        def scale_body(e, carry2):
            w1v = plsc.load_gather(w1b, [jnp.full((16,), e, jnp.int32)])
            w2v = plsc.load_gather(w2b, [jnp.full((16,), e, jnp.int32)])
            for v in range(8):
                cs = pl.ds(v * 16, 16)
                wv = w1v if v < 4 else w2v
                rows[p][e, cs] = rows[p][e, cs] * wv
            return carry2

        lax.fori_loop(0, _BLK, scale_body, 0)
        pltpu.sync_copy(rows[p], acc.at[dstall.at[b]], add=True)

    issue_gath(0, 0)

    def body2(i, carry):
        b = 2 * i
        issue_gath(b + 1, 1)
        compute(b, 0)
        issue_gath(b + 2, 0)
        compute(b + 1, 1)
        return carry

    lax.fori_loop(0, _NBLK // 2 - 1, body2, 0)
    issue_gath(_NBLK - 1, 1)
    compute(_NBLK - 2, 0)
    compute(_NBLK - 1, 1)
    plsc.subcore_barrier()

    rsl = pl.ds(s * _ROWS_PER_TILE, _ROWS_PER_TILE)
    pltpu.sync_copy(acc.at[rsl], out_hbm.at[c, rsl])


def _stage2_body(acc_ref, x_ref, x3_ref, g1_ref, be1_ref, g2_ref, be2_ref,
                 out_ref):
    A = acc_ref[0] + acc_ref[1]

    def lngat_out(nsl, dsl, g, be):
        o = A[:, nsl] / (A[:, dsl] + 1e-16)
        mu = jnp.mean(o, axis=1, keepdims=True)
        d = o - mu
        var = jnp.mean(d * d, axis=1, keepdims=True)
        return d / jnp.sqrt(var + 1e-5) * g + be

    x1 = jnp.maximum(lngat_out(slice(0, 63), slice(63, 64),
                               g1_ref[...], be1_ref[...]), 0.0)
    x2 = lngat_out(slice(64, 127), slice(127, 128), g2_ref[...], be2_ref[...])
    y = jnp.concatenate([x1, x2, x3_ref[...]], axis=1) + x_ref[...]
    m = jnp.max(y, axis=1, keepdims=True)
    z = y - m
    out_ref[...] = z - jnp.log(jnp.sum(jnp.exp(z), axis=1, keepdims=True))


def kernel(x, edge_index, W_mlp, b_mlp, W1, a1_src, a1_dst, g1, be1,
           W2, a2_src, a2_dst, g2, be2):
    f32 = jnp.float32
    xp = jnp.zeros((_NPAD, _H), f32).at[:_N, :_F_IN].set(x)
    wmp = jnp.zeros((_H, _H), f32).at[:_F_IN, :].set(W_mlp)
    b2 = b_mlp.reshape(1, _H)
    w1p = jnp.zeros((_H, 64), f32).at[:, :_C].set(W1)
    w2p = jnp.zeros((_H, 64), f32).at[:, :_C].set(W2)
    a1s = jnp.zeros((64, 1), f32).at[:_C, 0].set(a1_src)
    a1d = jnp.zeros((64, 1), f32).at[:_C, 0].set(a1_dst)
    a2s = jnp.zeros((64, 1), f32).at[:_C, 0].set(a2_src)
    a2d = jnp.zeros((64, 1), f32).at[:_C, 0].set(a2_dst)

    hcat, st, x3, mm = pl.pallas_call(
        _stage1_body,
        out_shape=[
            jax.ShapeDtypeStruct((_NPAD, _H), f32),
            jax.ShapeDtypeStruct((_NPAD, 16), f32),
            jax.ShapeDtypeStruct((_NPAD, 1), f32),
            jax.ShapeDtypeStruct((2, 16), f32),
        ],
    )(xp, wmp, b2, w1p, w2p, a1s, a1d, a2s, a2d)

    src = jnp.concatenate(
        [edge_index[0], jnp.zeros((_EPAD - _E,), jnp.int32)]
    ).reshape(_NW, _NBLK, _BLK)
    dst = jnp.concatenate(
        [edge_index[1], jnp.full((_EPAD - _E,), _N, jnp.int32)]
    ).reshape(_NW, _NBLK, _BLK)
    zrows = jnp.zeros((_ROWS_PER_TILE, _H), f32)

    mesh = plsc.VectorSubcoreMesh(core_axis_name="c", subcore_axis_name="s")
    acc = pl.kernel(
        _sc_body,
        mesh=mesh,
        out_type=jax.ShapeDtypeStruct((_NC, _NPAD, _H), f32),
        compiler_params=pltpu.CompilerParams(
            needs_layout_passes=False, use_tc_tiling_on_sc=False),
        scratch_types=[
            pltpu.VMEM((_NBLK, _BLK), jnp.int32),
            pltpu.VMEM((_NBLK, _BLK), jnp.int32),
            pltpu.VMEM((_BLK, 16), f32),
            pltpu.VMEM((_BLK, 16), f32),
            pltpu.VMEM((_BLK, 16), f32),
            pltpu.VMEM((_BLK, 16), f32),
            pltpu.VMEM((_BLK, _H), f32),
            pltpu.VMEM((_BLK, _H), f32),
            pltpu.VMEM((_BLK,), f32),
            pltpu.VMEM((_BLK,), f32),
            pltpu.VMEM((2, 16), f32),
            pltpu.VMEM_SHARED((_NPAD, _H), f32),
            pltpu.SemaphoreType.DMA,
            pltpu.SemaphoreType.DMA,
        ],
    )(hcat, st, src, dst, zrows, mm)

    xres = jnp.zeros((_NPAD, _F_IN), f32).at[:_N, :].set(x)
    out = pl.pallas_call(
        _stage2_body,
        out_shape=jax.ShapeDtypeStruct((_NPAD, _F_IN), f32),
    )(acc, xres, x3, g1.reshape(1, _C), be1.reshape(1, _C),
      g2.reshape(1, _C), be2.reshape(1, _C))
    return out[:_N]


# speedup vs baseline: 34.8912x; 1.1777x over previous
import functools

import jax
import jax.numpy as jnp
from jax import lax
from jax.experimental import pallas as pl
from jax.experimental.pallas import tpu as pltpu
from jax.experimental.pallas import tpu_sc as plsc

_N = 10000
_E = 320000
_F_IN = 127
_H = 128
_C = 63

_NC = 2
_NS = 16
_NW = _NC * _NS

_BLK = 80
_PER_TILE = 10080
_NBLK = _PER_TILE // _BLK
_EPAD = _PER_TILE * _NW
_NPAD = 10112
_ROWS_PER_TILE = _NPAD // _NS


def _stage1_body(x_ref, wm_ref, b_ref, w1_ref, w2_ref,
                 a1s_ref, a1d_ref, a2s_ref, a2d_ref,
                 hcat_ref, st_ref, x3_ref, mm_ref):
    x0 = jnp.maximum(
        jnp.dot(x_ref[...], wm_ref[...], preferred_element_type=jnp.float32)
        + b_ref[...], 0.0)
    h1 = jnp.dot(x0, w1_ref[...], preferred_element_type=jnp.float32)
    h2 = jnp.dot(x0, w2_ref[...], preferred_element_type=jnp.float32)
    col = lax.broadcasted_iota(jnp.int32, (_NPAD, 64), 1)
    h1c = jnp.where(col == 63, 1.0, h1)
    h2c = jnp.where(col == 63, 1.0, h2)
    hcat_ref[...] = jnp.concatenate([h1c, h2c], axis=1)
    s1s = jnp.dot(h1, a1s_ref[...], preferred_element_type=jnp.float32)
    s1d = jnp.dot(h1, a1d_ref[...], preferred_element_type=jnp.float32)
    s2s = jnp.dot(h2, a2s_ref[...], preferred_element_type=jnp.float32)
    s2d = jnp.dot(h2, a2d_ref[...], preferred_element_type=jnp.float32)
    st_ref[...] = jnp.concatenate(
        [s1s, s1d, s2s, s2d, jnp.zeros((_NPAD, 12), jnp.float32)], axis=1)
    x3_ref[...] = jnp.max(x0, axis=1, keepdims=True)
    m1 = jnp.max(s1s) + jnp.max(s1d)
    m2 = jnp.max(s2s) + jnp.max(s2d)
    mm_ref[...] = jnp.stack(
        [jnp.full((16,), m1, jnp.float32), jnp.full((16,), m2, jnp.float32)])


def _sc_body(hcat_hbm, st_hbm, src_hbm, dst_hbm, zrows_hbm, mm_hbm, out_hbm,
             srcall, dstall, ssb0, ssb1, sdb0, sdb1, rows0, rows1,
             w1b, w2b, m_v, acc, sem_g0, sem_g1):
    c = lax.axis_index("c")
    s = lax.axis_index("s")
    wid = s * _NC + c

    ssb = (ssb0, ssb1)
    sdb = (sdb0, sdb1)
    rows = (rows0, rows1)
    sem_g = (sem_g0, sem_g1)

    pltpu.sync_copy(mm_hbm, m_v)
    pltpu.sync_copy(src_hbm.at[wid], srcall)
    pltpu.sync_copy(dst_hbm.at[wid], dstall)
    pltpu.sync_copy(zrows_hbm, acc.at[pl.ds(s * _ROWS_PER_TILE, _ROWS_PER_TILE)])
    plsc.subcore_barrier()

    m1 = m_v[0]
    m2 = m_v[1]

    def issue_gath(b, p):
        sidx = srcall.at[b]
        didx = dstall.at[b]
        pltpu.async_copy(st_hbm.at[sidx], ssb[p], sem_g[p])
        pltpu.async_copy(st_hbm.at[didx], sdb[p], sem_g[p])
        pltpu.async_copy(hcat_hbm.at[sidx], rows[p], sem_g[p])

    def compute(b, p):
        pltpu.make_async_copy(st_hbm.at[pl.ds(0, _BLK)], ssb[p], sem_g[p]).wait()
        pltpu.make_async_copy(st_hbm.at[pl.ds(0, _BLK)], sdb[p], sem_g[p]).wait()
        pltpu.make_async_copy(
            hcat_hbm.at[pl.ds(0, _BLK)], rows[p], sem_g[p]).wait()
        for j in range(_BLK // 16):
            sl = pl.ds(j * 16, 16)
            e16 = j * 16 + lax.iota(jnp.int32, 16)
            ss1 = plsc.load_gather(ssb[p], [e16, jnp.full((16,), 0, jnp.int32)])
            sd1 = plsc.load_gather(sdb[p], [e16, jnp.full((16,), 1, jnp.int32)])
            t1 = ss1 + sd1
            e1 = jnp.maximum(t1, 0.2 * t1)
            w1b[sl] = jnp.exp(e1 - m1)
            ss2 = plsc.load_gather(ssb[p], [e16, jnp.full((16,), 2, jnp.int32)])
            sd2 = plsc.load_gather(sdb[p], [e16, jnp.full((16,), 3, jnp.int32)])
            t2 = ss2 + sd2
            e2 = jnp.maximum(t2, 0.2 * t2)
            w2b[sl] = jnp.exp(e2 - m2)

        @plsc.parallel_loop(0, _BLK, unroll=4)
        def scale_body(e):
            w1v = plsc.load_gather(w1b, [jnp.full((16,), e, jnp.int32)])
            w2v = plsc.load_gather(w2b, [jnp.full((16,), e, jnp.int32)])
            for v in range(8):
                cs = pl.ds(v * 16, 16)
                wv = w1v if v < 4 else w2v
                rows[p][e, cs] = rows[p][e, cs] * wv
        pltpu.sync_copy(rows[p], acc.at[dstall.at[b]], add=True)

    issue_gath(0, 0)

    def body2(i, carry):
        b = 2 * i
        issue_gath(b + 1, 1)
        compute(b, 0)
        issue_gath(b + 2, 0)
        compute(b + 1, 1)
        return carry

    lax.fori_loop(0, _NBLK // 2 - 1, body2, 0)
    issue_gath(_NBLK - 1, 1)
    compute(_NBLK - 2, 0)
    compute(_NBLK - 1, 1)
    plsc.subcore_barrier()

    rsl = pl.ds(s * _ROWS_PER_TILE, _ROWS_PER_TILE)
    pltpu.sync_copy(acc.at[rsl], out_hbm.at[c, rsl])


def _stage2_body(acc_ref, x_ref, x3_ref, g1_ref, be1_ref, g2_ref, be2_ref,
                 out_ref):
    A = acc_ref[0] + acc_ref[1]

    def lngat_out(nsl, dsl, g, be):
        o = A[:, nsl] / (A[:, dsl] + 1e-16)
        mu = jnp.mean(o, axis=1, keepdims=True)
        d = o - mu
        var = jnp.mean(d * d, axis=1, keepdims=True)
        return d / jnp.sqrt(var + 1e-5) * g + be

    x1 = jnp.maximum(lngat_out(slice(0, 63), slice(63, 64),
                               g1_ref[...], be1_ref[...]), 0.0)
    x2 = lngat_out(slice(64, 127), slice(127, 128), g2_ref[...], be2_ref[...])
    y = jnp.concatenate([x1, x2, x3_ref[...]], axis=1) + x_ref[...]
    m = jnp.max(y, axis=1, keepdims=True)
    z = y - m
    out_ref[...] = z - jnp.log(jnp.sum(jnp.exp(z), axis=1, keepdims=True))


def kernel(x, edge_index, W_mlp, b_mlp, W1, a1_src, a1_dst, g1, be1,
           W2, a2_src, a2_dst, g2, be2):
    f32 = jnp.float32
    xp = jnp.zeros((_NPAD, _H), f32).at[:_N, :_F_IN].set(x)
    wmp = jnp.zeros((_H, _H), f32).at[:_F_IN, :].set(W_mlp)
    b2 = b_mlp.reshape(1, _H)
    w1p = jnp.zeros((_H, 64), f32).at[:, :_C].set(W1)
    w2p = jnp.zeros((_H, 64), f32).at[:, :_C].set(W2)
    a1s = jnp.zeros((64, 1), f32).at[:_C, 0].set(a1_src)
    a1d = jnp.zeros((64, 1), f32).at[:_C, 0].set(a1_dst)
    a2s = jnp.zeros((64, 1), f32).at[:_C, 0].set(a2_src)
    a2d = jnp.zeros((64, 1), f32).at[:_C, 0].set(a2_dst)

    hcat, st, x3, mm = pl.pallas_call(
        _stage1_body,
        out_shape=[
            jax.ShapeDtypeStruct((_NPAD, _H), f32),
            jax.ShapeDtypeStruct((_NPAD, 16), f32),
            jax.ShapeDtypeStruct((_NPAD, 1), f32),
            jax.ShapeDtypeStruct((2, 16), f32),
        ],
    )(xp, wmp, b2, w1p, w2p, a1s, a1d, a2s, a2d)

    src = jnp.concatenate(
        [edge_index[0], jnp.zeros((_EPAD - _E,), jnp.int32)]
    ).reshape(_NW, _NBLK, _BLK)
    dst = jnp.concatenate(
        [edge_index[1], jnp.full((_EPAD - _E,), _N, jnp.int32)]
    ).reshape(_NW, _NBLK, _BLK)
    zrows = jnp.zeros((_ROWS_PER_TILE, _H), f32)

    mesh = plsc.VectorSubcoreMesh(core_axis_name="c", subcore_axis_name="s")
    acc = pl.kernel(
        _sc_body,
        mesh=mesh,
        out_type=jax.ShapeDtypeStruct((_NC, _NPAD, _H), f32),
        compiler_params=pltpu.CompilerParams(
            needs_layout_passes=False, use_tc_tiling_on_sc=False),
        scratch_types=[
            pltpu.VMEM((_NBLK, _BLK), jnp.int32),
            pltpu.VMEM((_NBLK, _BLK), jnp.int32),
            pltpu.VMEM((_BLK, 16), f32),
            pltpu.VMEM((_BLK, 16), f32),
            pltpu.VMEM((_BLK, 16), f32),
            pltpu.VMEM((_BLK, 16), f32),
            pltpu.VMEM((_BLK, _H), f32),
            pltpu.VMEM((_BLK, _H), f32),
            pltpu.VMEM((_BLK,), f32),
            pltpu.VMEM((_BLK,), f32),
            pltpu.VMEM((2, 16), f32),
            pltpu.VMEM_SHARED((_NPAD, _H), f32),
            pltpu.SemaphoreType.DMA,
            pltpu.SemaphoreType.DMA,
        ],
    )(hcat, st, src, dst, zrows, mm)

    xres = jnp.zeros((_NPAD, _F_IN), f32).at[:_N, :].set(x)
    out = pl.pallas_call(
        _stage2_body,
        out_shape=jax.ShapeDtypeStruct((_NPAD, _F_IN), f32),
    )(acc, xres, x3, g1.reshape(1, _C), be1.reshape(1, _C),
      g2.reshape(1, _C), be2.reshape(1, _C))
    return out[:_N]


# speedup vs baseline: 35.0121x; 1.0035x over previous
import functools

import jax
import jax.numpy as jnp
from jax import lax
from jax.experimental import pallas as pl
from jax.experimental.pallas import tpu as pltpu
from jax.experimental.pallas import tpu_sc as plsc

_N = 10000
_E = 320000
_F_IN = 127
_H = 128
_C = 63

_NC = 2
_NS = 16
_NW = _NC * _NS

_BLK = 80
_PER_TILE = 10080
_NBLK = _PER_TILE // _BLK
_EPAD = _PER_TILE * _NW
_NPAD = 10112
_ROWS_PER_TILE = _NPAD // _NS


def _stage1_body(x_ref, wm_ref, b_ref, w1_ref, w2_ref,
                 a1s_ref, a1d_ref, a2s_ref, a2d_ref,
                 hcat_ref, st_ref, x3_ref, mm_ref):
    x0 = jnp.maximum(
        jnp.dot(x_ref[...], wm_ref[...], preferred_element_type=jnp.float32)
        + b_ref[...], 0.0)
    h1 = jnp.dot(x0, w1_ref[...], preferred_element_type=jnp.float32)
    h2 = jnp.dot(x0, w2_ref[...], preferred_element_type=jnp.float32)
    col = lax.broadcasted_iota(jnp.int32, (_NPAD, 64), 1)
    h1c = jnp.where(col == 63, 1.0, h1)
    h2c = jnp.where(col == 63, 1.0, h2)
    hcat_ref[...] = jnp.concatenate([h1c, h2c], axis=1)
    s1s = jnp.dot(h1, a1s_ref[...], preferred_element_type=jnp.float32)
    s1d = jnp.dot(h1, a1d_ref[...], preferred_element_type=jnp.float32)
    s2s = jnp.dot(h2, a2s_ref[...], preferred_element_type=jnp.float32)
    s2d = jnp.dot(h2, a2d_ref[...], preferred_element_type=jnp.float32)
    st_ref[...] = jnp.concatenate(
        [s1s, s1d, s2s, s2d, jnp.zeros((_NPAD, 12), jnp.float32)], axis=1)
    x3_ref[...] = jnp.max(x0, axis=1, keepdims=True)
    m1 = jnp.max(s1s) + jnp.max(s1d)
    m2 = jnp.max(s2s) + jnp.max(s2d)
    mm_ref[...] = jnp.stack(
        [jnp.full((16,), m1, jnp.float32), jnp.full((16,), m2, jnp.float32)])


def _sc_body(hcat_hbm, st_hbm, src_hbm, dst_hbm, zrows_hbm, mm_hbm, out_hbm,
             srcall, dstall, ssb0, ssb1, sdb0, sdb1, rows0, rows1,
             w1b, w2b, m_v, acc, sem_g0, sem_g1):
    c = lax.axis_index("c")
    s = lax.axis_index("s")
    wid = s * _NC + c

    ssb = (ssb0, ssb1)
    sdb = (sdb0, sdb1)
    rows = (rows0, rows1)
    sem_g = (sem_g0, sem_g1)

    pltpu.sync_copy(mm_hbm, m_v)
    pltpu.sync_copy(src_hbm.at[wid], srcall)
    pltpu.sync_copy(dst_hbm.at[wid], dstall)
    pltpu.sync_copy(zrows_hbm, acc.at[pl.ds(s * _ROWS_PER_TILE, _ROWS_PER_TILE)])
    plsc.subcore_barrier()

    m1 = m_v[0]
    m2 = m_v[1]

    def issue_gath(b, p):
        sidx = srcall.at[b]
        didx = dstall.at[b]
        pltpu.async_copy(st_hbm.at[sidx], ssb[p], sem_g[p])
        pltpu.async_copy(st_hbm.at[didx], sdb[p], sem_g[p])
        pltpu.async_copy(hcat_hbm.at[sidx], rows[p], sem_g[p])

    def compute(b, p):
        pltpu.make_async_copy(st_hbm.at[pl.ds(0, _BLK)], ssb[p], sem_g[p]).wait()
        pltpu.make_async_copy(st_hbm.at[pl.ds(0, _BLK)], sdb[p], sem_g[p]).wait()
        pltpu.make_async_copy(
            hcat_hbm.at[pl.ds(0, _BLK)], rows[p], sem_g[p]).wait()
        @plsc.parallel_loop(0, _BLK // 16, unroll=_BLK // 16)
        def score_body(j):
            sl = pl.ds(j * 16, 16)
            e16 = j * 16 + lax.iota(jnp.int32, 16)
            ss1 = plsc.load_gather(ssb[p], [e16, jnp.full((16,), 0, jnp.int32)])
            sd1 = plsc.load_gather(sdb[p], [e16, jnp.full((16,), 1, jnp.int32)])
            t1 = ss1 + sd1
            e1 = jnp.maximum(t1, 0.2 * t1)
            w1b[sl] = jnp.exp(e1 - m1)
            ss2 = plsc.load_gather(ssb[p], [e16, jnp.full((16,), 2, jnp.int32)])
            sd2 = plsc.load_gather(sdb[p], [e16, jnp.full((16,), 3, jnp.int32)])
            t2 = ss2 + sd2
            e2 = jnp.maximum(t2, 0.2 * t2)
            w2b[sl] = jnp.exp(e2 - m2)

        @plsc.parallel_loop(0, _BLK, unroll=8)
        def scale_body(e):
            w1v = plsc.load_gather(w1b, [jnp.full((16,), e, jnp.int32)])
            w2v = plsc.load_gather(w2b, [jnp.full((16,), e, jnp.int32)])
            for v in range(8):
                cs = pl.ds(v * 16, 16)
                wv = w1v if v < 4 else w2v
                rows[p][e, cs] = rows[p][e, cs] * wv
        pltpu.sync_copy(rows[p], acc.at[dstall.at[b]], add=True)

    issue_gath(0, 0)

    def body2(i, carry):
        b = 2 * i
        issue_gath(b + 1, 1)
        compute(b, 0)
        issue_gath(b + 2, 0)
        compute(b + 1, 1)
        return carry

    lax.fori_loop(0, _NBLK // 2 - 1, body2, 0)
    issue_gath(_NBLK - 1, 1)
    compute(_NBLK - 2, 0)
    compute(_NBLK - 1, 1)
    plsc.subcore_barrier()

    rsl = pl.ds(s * _ROWS_PER_TILE, _ROWS_PER_TILE)
    pltpu.sync_copy(acc.at[rsl], out_hbm.at[c, rsl])


def _stage2_body(acc_ref, x_ref, x3_ref, g1_ref, be1_ref, g2_ref, be2_ref,
                 out_ref):
    A = acc_ref[0] + acc_ref[1]

    def lngat_out(nsl, dsl, g, be):
        o = A[:, nsl] / (A[:, dsl] + 1e-16)
        mu = jnp.mean(o, axis=1, keepdims=True)
        d = o - mu
        var = jnp.mean(d * d, axis=1, keepdims=True)
        return d / jnp.sqrt(var + 1e-5) * g + be

    x1 = jnp.maximum(lngat_out(slice(0, 63), slice(63, 64),
                               g1_ref[...], be1_ref[...]), 0.0)
    x2 = lngat_out(slice(64, 127), slice(127, 128), g2_ref[...], be2_ref[...])
    y = jnp.concatenate([x1, x2, x3_ref[...]], axis=1) + x_ref[...]
    m = jnp.max(y, axis=1, keepdims=True)
    z = y - m
    out_ref[...] = z - jnp.log(jnp.sum(jnp.exp(z), axis=1, keepdims=True))


def kernel(x, edge_index, W_mlp, b_mlp, W1, a1_src, a1_dst, g1, be1,
           W2, a2_src, a2_dst, g2, be2):
    f32 = jnp.float32
    xp = jnp.zeros((_NPAD, _H), f32).at[:_N, :_F_IN].set(x)
    wmp = jnp.zeros((_H, _H), f32).at[:_F_IN, :].set(W_mlp)
    b2 = b_mlp.reshape(1, _H)
    w1p = jnp.zeros((_H, 64), f32).at[:, :_C].set(W1)
    w2p = jnp.zeros((_H, 64), f32).at[:, :_C].set(W2)
    a1s = jnp.zeros((64, 1), f32).at[:_C, 0].set(a1_src)
    a1d = jnp.zeros((64, 1), f32).at[:_C, 0].set(a1_dst)
    a2s = jnp.zeros((64, 1), f32).at[:_C, 0].set(a2_src)
    a2d = jnp.zeros((64, 1), f32).at[:_C, 0].set(a2_dst)

    hcat, st, x3, mm = pl.pallas_call(
        _stage1_body,
        out_shape=[
            jax.ShapeDtypeStruct((_NPAD, _H), f32),
            jax.ShapeDtypeStruct((_NPAD, 16), f32),
            jax.ShapeDtypeStruct((_NPAD, 1), f32),
            jax.ShapeDtypeStruct((2, 16), f32),
        ],
    )(xp, wmp, b2, w1p, w2p, a1s, a1d, a2s, a2d)

    src = jnp.concatenate(
        [edge_index[0], jnp.zeros((_EPAD - _E,), jnp.int32)]
    ).reshape(_NW, _NBLK, _BLK)
    dst = jnp.concatenate(
        [edge_index[1], jnp.full((_EPAD - _E,), _N, jnp.int32)]
    ).reshape(_NW, _NBLK, _BLK)
    zrows = jnp.zeros((_ROWS_PER_TILE, _H), f32)

    mesh = plsc.VectorSubcoreMesh(core_axis_name="c", subcore_axis_name="s")
    acc = pl.kernel(
        _sc_body,
        mesh=mesh,
        out_type=jax.ShapeDtypeStruct((_NC, _NPAD, _H), f32),
        compiler_params=pltpu.CompilerParams(
            needs_layout_passes=False, use_tc_tiling_on_sc=False),
        scratch_types=[
            pltpu.VMEM((_NBLK, _BLK), jnp.int32),
            pltpu.VMEM((_NBLK, _BLK), jnp.int32),
            pltpu.VMEM((_BLK, 16), f32),
            pltpu.VMEM((_BLK, 16), f32),
            pltpu.VMEM((_BLK, 16), f32),
            pltpu.VMEM((_BLK, 16), f32),
            pltpu.VMEM((_BLK, _H), f32),
            pltpu.VMEM((_BLK, _H), f32),
            pltpu.VMEM((_BLK,), f32),
            pltpu.VMEM((_BLK,), f32),
            pltpu.VMEM((2, 16), f32),
            pltpu.VMEM_SHARED((_NPAD, _H), f32),
            pltpu.SemaphoreType.DMA,
            pltpu.SemaphoreType.DMA,
        ],
    )(hcat, st, src, dst, zrows, mm)

    xres = jnp.zeros((_NPAD, _F_IN), f32).at[:_N, :].set(x)
    out = pl.pallas_call(
        _stage2_body,
        out_shape=jax.ShapeDtypeStruct((_NPAD, _F_IN), f32),
    )(acc, xres, x3, g1.reshape(1, _C), be1.reshape(1, _C),
      g2.reshape(1, _C), be2.reshape(1, _C))
    return out[:_N]
